# Initial kernel scaffold; baseline (speedup 1.0000x reference)
import jax
import jax.numpy as jnp
from jax.experimental import pallas as pl


def kernel(x):
    raise NotImplementedError("write your pallas kernel here")


# speedup vs baseline: 13.2690x; 13.2690x over previous
import jax
import jax.numpy as jnp
from jax.experimental import pallas as pl
from jax.experimental.pallas import tpu as pltpu

_ROWS = 32
_COLS = 32768
_K = 512 * _ROWS


def _body(bits_ref, out_ref):
    bits = bits_ref[...]
    u = jax.lax.bitcast_convert_type(bits, jnp.uint32)
    key = jnp.where(bits >= 0, u | jnp.uint32(0x80000000), ~u)

    def key_step(_, carry):
        t, bit = carry
        cand = t | bit
        cnt = jnp.sum((key >= cand).astype(jnp.int32))
        t = jnp.where(cnt >= _K, cand, t)
        return t, bit >> jnp.uint32(1)

    T, _ = jax.lax.fori_loop(
        0, 32, key_step, (jnp.uint32(0), jnp.uint32(0x80000000))
    )

    m = jnp.sum((key > T).astype(jnp.int32))
    r = _K - m
    tie = key == T

    row = jax.lax.broadcasted_iota(jnp.int32, (_ROWS, _COLS), 0)
    col = jax.lax.broadcasted_iota(jnp.int32, (_ROWS, _COLS), 1)
    idx = row * _COLS + col

    def idx_step(_, carry):
        v, bit = carry
        cand = v + bit
        cnt = jnp.sum((tie & (idx < cand)).astype(jnp.int32))
        v = jnp.where(cnt < r, cand, v)
        return v, bit >> 1

    v, _ = jax.lax.fori_loop(
        0, 21, idx_step, (jnp.int32(0), jnp.int32(1 << 20))
    )

    keep = (key > T) | (tie & (idx <= v))
    out_ref[...] = jnp.where(keep, bits, jnp.int32(0))


def kernel(x):
    bits = jax.lax.bitcast_convert_type(x, jnp.int32)
    out_bits = pl.pallas_call(
        _body,
        out_shape=jax.ShapeDtypeStruct((_ROWS, _COLS), jnp.int32),
    )(bits)
    return jax.lax.bitcast_convert_type(out_bits, jnp.float32)
